# baseline (speedup 1.0000x reference)
import functools

import jax
import jax.numpy as jnp
from jax import lax
from jax.experimental import pallas as pl
from jax.experimental.pallas import tpu as pltpu
from jax.experimental.pallas import tpu_sc as plsc

_VOCAB = 1000000
_EMB = 64
_BATCH = 16384
_HIST = 200
_B = _BATCH * _HIST
_NW = 32
_BPW = _B // _NW
_CHUNK = 800
_NCHUNK = _BPW // _CHUNK


def _make_lookup():
    mesh = plsc.VectorSubcoreMesh(core_axis_name="c", subcore_axis_name="s")

    @functools.partial(
        pl.kernel,
        mesh=mesh,
        out_type=jax.ShapeDtypeStruct((_B, _EMB), jnp.float32),
        scratch_types=[
            pltpu.VMEM((_CHUNK,), jnp.int32),
            pltpu.VMEM((_CHUNK, _EMB), jnp.float32),
            pltpu.SemaphoreType.DMA,
        ],
        compiler_params=pltpu.CompilerParams(use_tc_tiling_on_sc=False),
    )
    def lookup(idx_hbm, table_hbm, out_hbm, idx_v, rows_v, sem):
        wid = lax.axis_index("s") * 2 + lax.axis_index("c")
        base = wid * _BPW

        def body(g, carry):
            off = base + g * _CHUNK
            pltpu.sync_copy(idx_hbm.at[pl.ds(off, _CHUNK)], idx_v)
            pltpu.async_copy(table_hbm.at[idx_v], rows_v, sem).wait()
            pltpu.sync_copy(rows_v, out_hbm.at[pl.ds(off, _CHUNK)])
            return carry

        lax.fori_loop(0, _NCHUNK, body, 0)

    return lookup


_lookup = _make_lookup()


@jax.jit
def kernel(indices, table):
    out = _lookup(indices.reshape(_B), table)
    return out.reshape(_BATCH, _HIST, _EMB)


# speedup vs baseline: 1.0486x; 1.0486x over previous
import functools

import jax
import jax.numpy as jnp
from jax import lax
from jax.experimental import pallas as pl
from jax.experimental.pallas import tpu as pltpu
from jax.experimental.pallas import tpu_sc as plsc

_VOCAB = 1000000
_EMB = 64
_BATCH = 16384
_HIST = 200
_B = _BATCH * _HIST
_NW = 32
_BPW = _B // _NW
_CHUNK = 800
_NCHUNK = _BPW // _CHUNK
_NBUF = 2


def _make_lookup():
    mesh = plsc.VectorSubcoreMesh(core_axis_name="c", subcore_axis_name="s")

    @functools.partial(
        pl.kernel,
        mesh=mesh,
        out_type=jax.ShapeDtypeStruct((_B, _EMB), jnp.float32),
        scratch_types=[
            pltpu.VMEM((_CHUNK,), jnp.int32),
            pltpu.VMEM((_CHUNK,), jnp.int32),
            pltpu.VMEM((_CHUNK, _EMB), jnp.float32),
            pltpu.VMEM((_CHUNK, _EMB), jnp.float32),
            pltpu.SemaphoreType.DMA,
            pltpu.SemaphoreType.DMA,
            pltpu.SemaphoreType.DMA,
            pltpu.SemaphoreType.DMA,
            pltpu.SemaphoreType.DMA,
            pltpu.SemaphoreType.DMA,
        ],
        compiler_params=pltpu.CompilerParams(use_tc_tiling_on_sc=False),
    )
    def lookup(idx_hbm, table_hbm, out_hbm, idx_v0, idx_v1, rows_v0, rows_v1,
               sem_i0, sem_i1, sem_g0, sem_g1, sem_s0, sem_s1):
        idx_v = (idx_v0, idx_v1)
        rows_v = (rows_v0, rows_v1)
        sem_i = (sem_i0, sem_i1)
        sem_g = (sem_g0, sem_g1)
        sem_s = (sem_s0, sem_s1)
        wid = lax.axis_index("s") * 2 + lax.axis_index("c")
        base = wid * _BPW

        for b in range(_NBUF):
            pltpu.async_copy(
                idx_hbm.at[pl.ds(base + b * _CHUNK, _CHUNK)],
                idx_v[b], sem_i[b])

        def body(i, carry):
            g0 = i * _NBUF
            for b in range(_NBUF):
                g = g0 + b
                off = base + g * _CHUNK
                pltpu.make_async_copy(
                    idx_hbm.at[pl.ds(off, _CHUNK)], idx_v[b],
                    sem_i[b]).wait()
                @pl.when(g0 > 0)
                def _():
                    pltpu.make_async_copy(
                        rows_v[b], out_hbm.at[pl.ds(base, _CHUNK)],
                        sem_s[b]).wait()
                pltpu.async_copy(table_hbm.at[idx_v[b]], rows_v[b],
                                 sem_g[b])
                pltpu.make_async_copy(table_hbm.at[idx_v[b]],
                                      rows_v[b], sem_g[b]).wait()
                @pl.when(g + _NBUF < _NCHUNK)
                def _():
                    pltpu.async_copy(
                        idx_hbm.at[pl.ds(off + _NBUF * _CHUNK, _CHUNK)],
                        idx_v[b], sem_i[b])
                pltpu.async_copy(rows_v[b],
                                 out_hbm.at[pl.ds(off, _CHUNK)], sem_s[b])
            return carry

        lax.fori_loop(0, _NCHUNK // _NBUF, body, 0)

        for b in range(_NBUF):
            pltpu.make_async_copy(
                rows_v[b], out_hbm.at[pl.ds(base, _CHUNK)],
                sem_s[b]).wait()

    return lookup


_lookup = _make_lookup()


@jax.jit
def kernel(indices, table):
    out = _lookup(indices.reshape(_B), table)
    return out.reshape(_BATCH, _HIST, _EMB)


# speedup vs baseline: 1.3535x; 1.2908x over previous
import functools

import jax
import jax.numpy as jnp
from jax import lax
from jax.experimental import pallas as pl
from jax.experimental.pallas import tpu as pltpu
from jax.experimental.pallas import tpu_sc as plsc

_VOCAB = 1000000
_EMB = 64
_PAD = 128
_BATCH = 16384
_HIST = 200
_B = _BATCH * _HIST
_NW = 32
_BPW = _B // _NW
_CHUNK = 400
_NCHUNK = _BPW // _CHUNK
_NBUF = 2
_QB = _CHUNK // _HIST


def _make_lookup():
    mesh = plsc.VectorSubcoreMesh(core_axis_name="c", subcore_axis_name="s")

    @functools.partial(
        pl.kernel,
        mesh=mesh,
        out_type=jax.ShapeDtypeStruct((_BATCH, _HIST, _PAD), jnp.float32),
        scratch_types=[
            pltpu.VMEM((_CHUNK,), jnp.int32),
            pltpu.VMEM((_CHUNK,), jnp.int32),
            pltpu.VMEM((_CHUNK, _PAD), jnp.float32),
            pltpu.VMEM((_CHUNK, _PAD), jnp.float32),
            pltpu.SemaphoreType.DMA,
            pltpu.SemaphoreType.DMA,
            pltpu.SemaphoreType.DMA,
            pltpu.SemaphoreType.DMA,
            pltpu.SemaphoreType.DMA,
            pltpu.SemaphoreType.DMA,
        ],
        compiler_params=pltpu.CompilerParams(use_tc_tiling_on_sc=True),
    )
    def lookup(idx_hbm, table_hbm, out_hbm, idx_v0, idx_v1, rows_v0, rows_v1,
               sem_i0, sem_i1, sem_g0, sem_g1, sem_s0, sem_s1):
        idx_v = (idx_v0, idx_v1)
        rows_v = (rows_v0, rows_v1)
        sem_i = (sem_i0, sem_i1)
        sem_g = (sem_g0, sem_g1)
        sem_s = (sem_s0, sem_s1)
        wid = lax.axis_index("s") * 2 + lax.axis_index("c")
        base = wid * _BPW

        for b in range(_NBUF):
            pltpu.async_copy(
                idx_hbm.at[pl.ds(base + b * _CHUNK, _CHUNK)],
                idx_v[b], sem_i[b])

        def body(i, carry):
            g0 = i * _NBUF
            for b in range(_NBUF):
                g = g0 + b
                off = base + g * _CHUNK
                pltpu.make_async_copy(
                    idx_hbm.at[pl.ds(off, _CHUNK)], idx_v[b],
                    sem_i[b]).wait()
                @pl.when(g0 > 0)
                def _():
                    for q in range(_QB):
                        pltpu.make_async_copy(
                            rows_v[b].at[pl.ds(q * _HIST, _HIST)],
                            out_hbm.at[0], sem_s[b]).wait()
                pltpu.async_copy(table_hbm.at[idx_v[b]], rows_v[b],
                                 sem_g[b])
                pltpu.make_async_copy(table_hbm.at[idx_v[b]],
                                      rows_v[b], sem_g[b]).wait()
                @pl.when(g + _NBUF < _NCHUNK)
                def _():
                    pltpu.async_copy(
                        idx_hbm.at[pl.ds(off + _NBUF * _CHUNK, _CHUNK)],
                        idx_v[b], sem_i[b])
                bat0 = off // _HIST
                for q in range(_QB):
                    pltpu.async_copy(
                        rows_v[b].at[pl.ds(q * _HIST, _HIST)],
                        out_hbm.at[bat0 + q], sem_s[b])
            return carry

        lax.fori_loop(0, _NCHUNK // _NBUF, body, 0)

        for b in range(_NBUF):
            for q in range(_QB):
                pltpu.make_async_copy(
                    rows_v[b].at[pl.ds(q * _HIST, _HIST)],
                    out_hbm.at[0], sem_s[b]).wait()

    return lookup


_lookup = _make_lookup()


@jax.jit
def kernel(indices, table):
    table_p = jnp.pad(table, ((0, 0), (0, _PAD - _EMB)))
    out = _lookup(indices.reshape(_B), table_p)
    return out[:, :, :_EMB]
